# baseline (speedup 1.0000x reference)
import functools

import jax
import jax.numpy as jnp
from jax import lax
from jax.experimental import pallas as pl
from jax.experimental.pallas import tpu as pltpu
from jax.experimental.pallas import tpu_sc as plsc

N_NODES = 10000
N_EDGES = 320000
D = 128
NUM_BASES = 4
NUM_RATINGS = 5

NC = 2
NS = 16
NW = NC * NS
CHUNK = 80
NCHUNKS = 128
EPW = NCHUNKS * CHUNK
E_PAD = NW * EPW
NBUF = 4
IDXR = 8
N_PAD = 10240
ROWS_PER_TILE = N_PAD // NS


def _i32(v):
    return jnp.asarray(v, jnp.int32)


def _table_body(a_ref, x_ref, bs_ref, src_ref, dst_in_ref, et_ref, out_ref,
                gidx_ref, dst_ref):
    i = pl.program_id(0)
    xb = x_ref[...]
    for r in range(NUM_RATINGS):
        w = a_ref[r, 0] * bs_ref[0]
        for k in range(1, NUM_BASES):
            w = w + a_ref[r, k] * bs_ref[k]
        out_ref[r] = lax.dot_general(
            xb, w, (((1,), (1,)), ((), ())),
            preferred_element_type=jnp.float32)

    pad = (E_PAD - N_EDGES) // D

    @pl.when(i == 0)
    def _():
        f = (lax.broadcasted_iota(jnp.int32, (pad, D), 0) * jnp.int32(D)
             + lax.broadcasted_iota(jnp.int32, (pad, D), 1))
        g = et_ref[...] * jnp.int32(N_NODES) + src_ref[...]
        gidx_ref[...] = jnp.concatenate([g, f], axis=0)
        dst_ref[...] = jnp.concatenate(
            [dst_in_ref[...],
             jnp.int32(N_NODES) + f % jnp.int32(N_PAD - N_NODES)], axis=0)


def _combine_body(x_ref, w0_ref, p_ref, out_ref):
    out_ref[...] = (
        lax.dot_general(x_ref[...], w0_ref[...], (((1,), (1,)), ((), ())),
                        preferred_element_type=jnp.float32)
        + p_ref[0] + p_ref[1])


def _sc_body(table, gidx1, dst1, out, gall, dall, rows0, rows1, rows2,
             rows3, acc, gsem, ssem, isem):
    i32 = jnp.int32
    rows = (rows0, rows1, rows2, rows3)
    c = lax.axis_index("c")
    s = lax.axis_index("s")
    wid = s * i32(NC) + c
    ebase = wid * i32(EPW)
    sbase = s * i32(ROWS_PER_TILE)

    def zfill(i, carry):
        for j in range(D // 16):
            rows0[i, pl.ds(j * 16, 16)] = jnp.zeros((16,), jnp.float32)
        return carry
    lax.fori_loop(i32(0), i32(CHUNK), zfill, 0)
    for t in range(ROWS_PER_TILE // CHUNK):
        pltpu.sync_copy(rows0, acc.at[pl.ds(sbase + i32(t * CHUNK), CHUNK)])
    rem = ROWS_PER_TILE % CHUNK
    if rem:
        pltpu.sync_copy(
            rows0.at[pl.ds(0, rem)],
            acc.at[pl.ds(sbase + i32(ROWS_PER_TILE - rem), rem)])
    plsc.subcore_barrier()

    def idx_start(k, m):
        off = ebase + k * i32(CHUNK)
        pltpu.async_copy(gidx1.at[pl.ds(off, CHUNK)],
                         gall.at[i32(m), i32(0)], isem)
        pltpu.async_copy(dst1.at[pl.ds(off, CHUNK)],
                         dall.at[i32(m), i32(0)], isem)

    def idx_wait(k, m):
        off = ebase + k * i32(CHUNK)
        pltpu.make_async_copy(gidx1.at[pl.ds(off, CHUNK)],
                              gall.at[i32(m), i32(0)], isem).wait()
        pltpu.make_async_copy(dst1.at[pl.ds(off, CHUNK)],
                              dall.at[i32(m), i32(0)], isem).wait()

    def gat_start(k, m, b):
        pltpu.async_copy(table.at[gall.at[i32(m), i32(0)]], rows[b], gsem)

    def gat_wait(k, m, b):
        pltpu.make_async_copy(table.at[gall.at[i32(m), i32(0)]], rows[b],
                              gsem).wait()

    def scat_start(k, m, b):
        pltpu.async_copy(rows[b], acc.at[dall.at[i32(m), i32(0)]], ssem,
                         add=True)

    def scat_wait(k, m, b):
        pltpu.make_async_copy(rows[b], acc.at[dall.at[i32(m), i32(0)]],
                              ssem).wait()

    for k in range(5):
        idx_start(i32(k), k % IDXR)
    for k in range(3):
        idx_wait(i32(k), k % IDXR)
        gat_start(i32(k), k % IDXR, k % NBUF)

    def main(t, carry):
        for u in range(IDXR):
            k = t * i32(IDXR) + i32(u)

            def kk(d):
                return k + i32(d)

            if True:
                @pl.when(kk(0) >= i32(1))
                def _():
                    scat_wait(kk(-1), (u - 1) % IDXR, (u - 1) % NBUF)

                @pl.when(kk(3) < i32(NCHUNKS))
                def _():
                    idx_wait(kk(3), (u + 3) % IDXR)
                    gat_start(kk(3), (u + 3) % IDXR, (u + 3) % NBUF)

                gat_wait(kk(0), u % IDXR, u % NBUF)
                scat_start(kk(0), u % IDXR, u % NBUF)

                @pl.when(kk(5) < i32(NCHUNKS))
                def _():
                    idx_start(kk(5), (u + 5) % IDXR)
        return carry
    lax.fori_loop(i32(0), i32(NCHUNKS // IDXR), main, 0)
    scat_wait(i32(NCHUNKS - 1), (NCHUNKS - 1) % IDXR, (NCHUNKS - 1) % NBUF)
    plsc.subcore_barrier()

    pltpu.sync_copy(acc.at[pl.ds(sbase, ROWS_PER_TILE)],
                    out.at[c, pl.ds(sbase, ROWS_PER_TILE)])


def kernel(x, Bs, A, W0, edge_index, edge_type):
    nb = 5
    blk = N_NODES // nb
    erows = N_EDGES // D
    src32 = edge_index[0].astype(jnp.int32).reshape(erows, D)
    dst32 = edge_index[1].astype(jnp.int32).reshape(erows, D)
    et32 = edge_type.astype(jnp.int32).reshape(erows, D)

    table, gidx, dst = pl.pallas_call(
        _table_body,
        grid=(nb,),
        in_specs=[
            pl.BlockSpec((NUM_RATINGS, NUM_BASES),
                         lambda i: (_i32(0), _i32(0)),
                         memory_space=pltpu.SMEM),
            pl.BlockSpec((blk, D), lambda i: (_i32(i), _i32(0))),
            pl.BlockSpec((NUM_BASES, D, D),
                         lambda i: (_i32(0), _i32(0), _i32(0))),
            pl.BlockSpec((erows, D), lambda i: (_i32(0), _i32(0))),
            pl.BlockSpec((erows, D), lambda i: (_i32(0), _i32(0))),
            pl.BlockSpec((erows, D), lambda i: (_i32(0), _i32(0))),
        ],
        out_specs=[
            pl.BlockSpec((NUM_RATINGS, blk, D),
                         lambda i: (_i32(0), _i32(i), _i32(0))),
            pl.BlockSpec((E_PAD // D, D), lambda i: (_i32(0), _i32(0))),
            pl.BlockSpec((E_PAD // D, D), lambda i: (_i32(0), _i32(0))),
        ],
        out_shape=[
            jax.ShapeDtypeStruct((NUM_RATINGS, N_NODES, D), jnp.float32),
            jax.ShapeDtypeStruct((E_PAD // D, D), jnp.int32),
            jax.ShapeDtypeStruct((E_PAD // D, D), jnp.int32),
        ],
    )(A, x, Bs, src32, dst32, et32)

    mesh = plsc.VectorSubcoreMesh(core_axis_name="c", subcore_axis_name="s")
    partials = pl.kernel(
        _sc_body,
        out_type=jax.ShapeDtypeStruct((NC, N_PAD, D), jnp.float32),
        mesh=mesh,
        scratch_types=[
            pltpu.VMEM((IDXR, 1, CHUNK), jnp.int32),
            pltpu.VMEM((IDXR, 1, CHUNK), jnp.int32),
            pltpu.VMEM((CHUNK, D), jnp.float32),
            pltpu.VMEM((CHUNK, D), jnp.float32),
            pltpu.VMEM((CHUNK, D), jnp.float32),
            pltpu.VMEM((CHUNK, D), jnp.float32),
            pltpu.VMEM_SHARED((N_PAD, D), jnp.float32),
            pltpu.SemaphoreType.DMA,
            pltpu.SemaphoreType.DMA,
            pltpu.SemaphoreType.DMA,
        ],
    )(table.reshape(NUM_RATINGS * N_NODES, D), gidx.reshape(E_PAD),
      dst.reshape(E_PAD))

    out = pl.pallas_call(
        _combine_body,
        grid=(nb,),
        in_specs=[
            pl.BlockSpec((blk, D), lambda i: (_i32(i), _i32(0))),
            pl.BlockSpec((D, D), lambda i: (_i32(0), _i32(0))),
            pl.BlockSpec((NC, blk, D), lambda i: (_i32(0), _i32(i), _i32(0))),
        ],
        out_specs=pl.BlockSpec((blk, D), lambda i: (_i32(i), _i32(0))),
        out_shape=jax.ShapeDtypeStruct((N_NODES, D), jnp.float32),
    )(x, W0, partials)
    return out


# speedup vs baseline: 1.0666x; 1.0666x over previous
import functools

import jax
import jax.numpy as jnp
from jax import lax
from jax.experimental import pallas as pl
from jax.experimental.pallas import tpu as pltpu
from jax.experimental.pallas import tpu_sc as plsc

N_NODES = 10000
N_EDGES = 320000
D = 128
NUM_BASES = 4
NUM_RATINGS = 5

NC = 2
NS = 16
NW = NC * NS
CHUNK = 80
NCHUNKS = 128
EPW = NCHUNKS * CHUNK
E_PAD = NW * EPW
NBUF = 4
IDXR = 8
N_PAD = 10240
ROWS_PER_TILE = N_PAD // NS


def _i32(v):
    return jnp.asarray(v, jnp.int32)


def _table_body(a_ref, x_ref, bs_ref, ei_ref, et_ref, out_ref, gidx_ref,
                dst_ref):
    i = pl.program_id(0)
    xb = x_ref[...]
    for r in range(NUM_RATINGS):
        w = a_ref[r, 0] * bs_ref[0]
        for k in range(1, NUM_BASES):
            w = w + a_ref[r, k] * bs_ref[k]
        out_ref[r] = lax.dot_general(
            xb, w, (((1,), (1,)), ((), ())),
            preferred_element_type=jnp.float32)

    pad = (E_PAD - N_EDGES) // D

    @pl.when(i == 0)
    def _():
        f = (lax.broadcasted_iota(jnp.int32, (pad, D), 0) * jnp.int32(D)
             + lax.broadcasted_iota(jnp.int32, (pad, D), 1))
        g = et_ref[...] * jnp.int32(N_NODES) + ei_ref[0]
        gidx_ref[...] = jnp.concatenate([g, f], axis=0)
        dst_ref[...] = jnp.concatenate(
            [ei_ref[1],
             jnp.int32(N_NODES) + f % jnp.int32(N_PAD - N_NODES)], axis=0)


def _combine_body(x_ref, w0_ref, p_ref, out_ref):
    out_ref[...] = (
        lax.dot_general(x_ref[...], w0_ref[...], (((1,), (1,)), ((), ())),
                        preferred_element_type=jnp.float32)
        + p_ref[0] + p_ref[1])


def _sc_body(table, gidx1, dst1, out, gall, dall, rows0, rows1, rows2,
             rows3, acc, gsem, ssem, isem):
    i32 = jnp.int32
    rows = (rows0, rows1, rows2, rows3)
    c = lax.axis_index("c")
    s = lax.axis_index("s")
    wid = s * i32(NC) + c
    ebase = wid * i32(EPW)
    sbase = s * i32(ROWS_PER_TILE)

    def zfill(i, carry):
        for j in range(D // 16):
            rows0[i, pl.ds(j * 16, 16)] = jnp.zeros((16,), jnp.float32)
        return carry
    lax.fori_loop(i32(0), i32(CHUNK), zfill, 0)
    for t in range(ROWS_PER_TILE // CHUNK):
        pltpu.sync_copy(rows0, acc.at[pl.ds(sbase + i32(t * CHUNK), CHUNK)])
    rem = ROWS_PER_TILE % CHUNK
    if rem:
        pltpu.sync_copy(
            rows0.at[pl.ds(0, rem)],
            acc.at[pl.ds(sbase + i32(ROWS_PER_TILE - rem), rem)])
    plsc.subcore_barrier()

    def idx_start(k, m):
        off = ebase + k * i32(CHUNK)
        pltpu.async_copy(gidx1.at[pl.ds(off, CHUNK)],
                         gall.at[i32(m), i32(0)], isem)
        pltpu.async_copy(dst1.at[pl.ds(off, CHUNK)],
                         dall.at[i32(m), i32(0)], isem)

    def idx_wait(k, m):
        off = ebase + k * i32(CHUNK)
        pltpu.make_async_copy(gidx1.at[pl.ds(off, CHUNK)],
                              gall.at[i32(m), i32(0)], isem).wait()
        pltpu.make_async_copy(dst1.at[pl.ds(off, CHUNK)],
                              dall.at[i32(m), i32(0)], isem).wait()

    def gat_start(k, m, b):
        pltpu.async_copy(table.at[gall.at[i32(m), i32(0)]], rows[b], gsem)

    def gat_wait(k, m, b):
        pltpu.make_async_copy(table.at[gall.at[i32(m), i32(0)]], rows[b],
                              gsem).wait()

    def scat_start(k, m, b):
        pltpu.async_copy(rows[b], acc.at[dall.at[i32(m), i32(0)]], ssem,
                         add=True)

    def scat_wait(k, m, b):
        pltpu.make_async_copy(rows[b], acc.at[dall.at[i32(m), i32(0)]],
                              ssem).wait()

    for k in range(5):
        idx_start(i32(k), k % IDXR)
    for k in range(3):
        idx_wait(i32(k), k % IDXR)
        gat_start(i32(k), k % IDXR, k % NBUF)

    def main(t, carry):
        for u in range(IDXR):
            k = t * i32(IDXR) + i32(u)

            def kk(d):
                return k + i32(d)

            if True:
                @pl.when(kk(0) >= i32(1))
                def _():
                    scat_wait(kk(-1), (u - 1) % IDXR, (u - 1) % NBUF)

                @pl.when(kk(3) < i32(NCHUNKS))
                def _():
                    idx_wait(kk(3), (u + 3) % IDXR)
                    gat_start(kk(3), (u + 3) % IDXR, (u + 3) % NBUF)

                gat_wait(kk(0), u % IDXR, u % NBUF)
                scat_start(kk(0), u % IDXR, u % NBUF)

                @pl.when(kk(5) < i32(NCHUNKS))
                def _():
                    idx_start(kk(5), (u + 5) % IDXR)
        return carry
    lax.fori_loop(i32(0), i32(NCHUNKS // IDXR), main, 0)
    scat_wait(i32(NCHUNKS - 1), (NCHUNKS - 1) % IDXR, (NCHUNKS - 1) % NBUF)
    plsc.subcore_barrier()

    pltpu.sync_copy(acc.at[pl.ds(sbase, ROWS_PER_TILE)],
                    out.at[c, pl.ds(sbase, ROWS_PER_TILE)])


def kernel(x, Bs, A, W0, edge_index, edge_type):
    nb = 5
    blk = N_NODES // nb
    erows = N_EDGES // D
    ei32 = edge_index.astype(jnp.int32)
    et32 = edge_type.astype(jnp.int32)

    table, gidx, dst = pl.pallas_call(
        _table_body,
        grid=(nb,),
        in_specs=[
            pl.BlockSpec((NUM_RATINGS, NUM_BASES),
                         lambda i: (_i32(0), _i32(0)),
                         memory_space=pltpu.SMEM),
            pl.BlockSpec((blk, D), lambda i: (_i32(i), _i32(0))),
            pl.BlockSpec((NUM_BASES, D, D),
                         lambda i: (_i32(0), _i32(0), _i32(0))),
            pl.BlockSpec((2, erows, D),
                         lambda i: (_i32(0), _i32(0), _i32(0))),
            pl.BlockSpec((erows, D), lambda i: (_i32(0), _i32(0))),
        ],
        out_specs=[
            pl.BlockSpec((NUM_RATINGS, blk, D),
                         lambda i: (_i32(0), _i32(i), _i32(0))),
            pl.BlockSpec((E_PAD // D, D), lambda i: (_i32(0), _i32(0))),
            pl.BlockSpec((E_PAD // D, D), lambda i: (_i32(0), _i32(0))),
        ],
        out_shape=[
            jax.ShapeDtypeStruct((NUM_RATINGS, N_NODES, D), jnp.float32),
            jax.ShapeDtypeStruct((E_PAD // D, D), jnp.int32),
            jax.ShapeDtypeStruct((E_PAD // D, D), jnp.int32),
        ],
    )(A, x, Bs, ei32.reshape(2, erows, D), et32.reshape(erows, D))

    mesh = plsc.VectorSubcoreMesh(core_axis_name="c", subcore_axis_name="s")
    partials = pl.kernel(
        _sc_body,
        out_type=jax.ShapeDtypeStruct((NC, N_PAD, D), jnp.float32),
        mesh=mesh,
        scratch_types=[
            pltpu.VMEM((IDXR, 1, CHUNK), jnp.int32),
            pltpu.VMEM((IDXR, 1, CHUNK), jnp.int32),
            pltpu.VMEM((CHUNK, D), jnp.float32),
            pltpu.VMEM((CHUNK, D), jnp.float32),
            pltpu.VMEM((CHUNK, D), jnp.float32),
            pltpu.VMEM((CHUNK, D), jnp.float32),
            pltpu.VMEM_SHARED((N_PAD, D), jnp.float32),
            pltpu.SemaphoreType.DMA,
            pltpu.SemaphoreType.DMA,
            pltpu.SemaphoreType.DMA,
        ],
    )(table.reshape(NUM_RATINGS * N_NODES, D), gidx.reshape(E_PAD),
      dst.reshape(E_PAD))

    out = pl.pallas_call(
        _combine_body,
        grid=(nb,),
        in_specs=[
            pl.BlockSpec((blk, D), lambda i: (_i32(i), _i32(0))),
            pl.BlockSpec((D, D), lambda i: (_i32(0), _i32(0))),
            pl.BlockSpec((NC, blk, D), lambda i: (_i32(0), _i32(i), _i32(0))),
        ],
        out_specs=pl.BlockSpec((blk, D), lambda i: (_i32(i), _i32(0))),
        out_shape=jax.ShapeDtypeStruct((N_NODES, D), jnp.float32),
    )(x, W0, partials)
    return out
